# baseline (speedup 1.0000x reference)
import jax
import jax.numpy as jnp
from jax.experimental import pallas as pl
from jax.experimental.pallas import tpu as pltpu

_BM = 512


def _readout_matmul_kernel(a_ref, w_ref, o_ref, wbf_ref):
    @pl.when(pl.program_id(0) == 0)
    def _cast_weight_once():
        wbf_ref[...] = w_ref[...].astype(jnp.bfloat16)

    a = a_ref[...].astype(jnp.bfloat16)
    o_ref[...] = jax.lax.dot_general(
        a, wbf_ref[...],
        dimension_numbers=(((1,), (1,)), ((), ())),
        preferred_element_type=jnp.float32,
    )


def kernel(embed, emb_weight):
    m, d = embed.shape
    l, _ = emb_weight.shape
    grid = (m // _BM,)
    return pl.pallas_call(
        _readout_matmul_kernel,
        grid=grid,
        in_specs=[
            pl.BlockSpec((_BM, d), lambda i: (i, 0)),
            pl.BlockSpec((l, d), lambda i: (0, 0)),
        ],
        out_specs=pl.BlockSpec((_BM, l), lambda i: (i, 0)),
        out_shape=jax.ShapeDtypeStruct((m, l), jnp.float32),
        scratch_shapes=[pltpu.VMEM((l, d), jnp.bfloat16)],
    )(embed, emb_weight)


# speedup vs baseline: 1.0139x; 1.0139x over previous
import jax
import jax.numpy as jnp
from jax.experimental import pallas as pl
from jax.experimental.pallas import tpu as pltpu

_BM = 1024


def _readout_matmul_kernel(a_ref, w_ref, o_ref):
    a = a_ref[...].astype(jnp.bfloat16)
    w = w_ref[...].astype(jnp.bfloat16)
    o_ref[...] = jax.lax.dot_general(
        a, w,
        dimension_numbers=(((1,), (1,)), ((), ())),
        preferred_element_type=jnp.float32,
    )


def kernel(embed, emb_weight):
    m, d = embed.shape
    l, _ = emb_weight.shape
    grid = (m // _BM,)
    return pl.pallas_call(
        _readout_matmul_kernel,
        grid=grid,
        in_specs=[
            pl.BlockSpec((_BM, d), lambda i: (i, 0)),
            pl.BlockSpec((l, d), lambda i: (0, 0)),
        ],
        out_specs=pl.BlockSpec((_BM, l), lambda i: (i, 0)),
        out_shape=jax.ShapeDtypeStruct((m, l), jnp.float32),
    )(embed, emb_weight)
